# baseline (speedup 1.0000x reference)
import functools

import jax
import jax.numpy as jnp
from jax import lax
from jax.experimental import pallas as pl
from jax.experimental.pallas import tpu as pltpu
from jax.experimental.pallas import tpu_sc as plsc

B = 4096
F_FIELDS = 26
E = 64
HIST = 20
G = 0.8
EMB = F_FIELDS * E
NB = B * F_FIELDS

NC = 2
NS = 16
NW = NC * NS
BAGS_W = NB // NW

NBH = NB // 2
BAGS_WH = NBH // NW
C = 52
CH = C * HIST
CHUNKS = BAGS_WH // C
UC = 832
UCHUNKS = BAGS_W // UC

LB = 32


def _tree_sum(vals):
    while len(vals) > 1:
        nxt = [a + b for a, b in zip(vals[0::2], vals[1::2])]
        if len(vals) % 2:
            nxt.append(vals[-1])
        vals = nxt
    return vals[0]


def _mesh():
    return plsc.VectorSubcoreMesh(core_axis_name="c", subcore_axis_name="s")


def _sc_item(iidx, item_table):
    @functools.partial(
        pl.kernel,
        mesh=_mesh(),
        compiler_params=pltpu.CompilerParams(use_tc_tiling_on_sc=False),
        out_type=jax.ShapeDtypeStruct((NBH, E), jnp.bfloat16),
        scratch_types=[
            pltpu.VMEM((CH,), jnp.int32),
            pltpu.VMEM((CH,), jnp.int32),
            pltpu.VMEM((CH,), jnp.int32),
            pltpu.VMEM((CH,), jnp.int32),
            pltpu.VMEM((CH, E), jnp.bfloat16),
            pltpu.VMEM((CH, E), jnp.bfloat16),
            pltpu.VMEM((C, E), jnp.bfloat16),
            pltpu.SemaphoreType.DMA,
            pltpu.SemaphoreType.DMA,
            pltpu.SemaphoreType.DMA,
            pltpu.SemaphoreType.DMA,
            pltpu.SemaphoreType.DMA,
            pltpu.SemaphoreType.DMA,
        ],
    )
    def k(iidx_hbm, itab_hbm, b_out, i0, i1, i2, i3, rows0, rows1, acc_v,
          is0, is1, is2, is3, gs0, gs1):
        wid = lax.axis_index("s") * NC + lax.axis_index("c")
        bag_base = wid * BAGS_WH

        idx_bufs = (i0, i1, i2, i3)
        idx_sems = (is0, is1, is2, is3)
        row_bufs = (rows0, rows1)
        row_sems = (gs0, gs1)

        def idx_copy(chunk, q):
            return pltpu.make_async_copy(
                iidx_hbm.at[pl.ds((bag_base + chunk * C) * HIST, CH)],
                idx_bufs[q], idx_sems[q])

        def gather(q, p):
            return pltpu.make_async_copy(itab_hbm.at[idx_bufs[q]],
                                         row_bufs[p], row_sems[p])

        def reduce_chunk(chunk, p):
            rows_v = row_bufs[p]

            def bag_body(c, carry):
                r0 = c * HIST
                for j in range(E // LB):
                    s = pl.ds(j * LB, LB)
                    acc_v[c, s] = _tree_sum(
                        [rows_v[r0 + h, s] for h in range(HIST)])
                return carry

            lax.fori_loop(0, C, bag_body, 0)
            pltpu.sync_copy(acc_v, b_out.at[pl.ds(bag_base + chunk * C, C)])

        for q in range(4):
            idx_copy(q, q).start()
        idx_copy(0, 0).wait()
        gather(0, 0).start()

        def quad_body(t, carry):
            for i in range(4):
                c = 4 * t + i
                q = i
                p = i % 2
                gather(q, p).wait()

                @pl.when(c + 4 < CHUNKS)
                def _():
                    idx_copy(c + 4, q).start()

                @pl.when(c + 1 < CHUNKS)
                def _():
                    idx_copy(c + 1, (q + 1) % 4).wait()
                    gather((q + 1) % 4, (p + 1) % 2).start()

                reduce_chunk(c, p)
            return carry

        lax.fori_loop(0, CHUNKS // 4, quad_body, 0)

    return k(iidx, item_table)


def _sc_user(uidx, user_table):
    @functools.partial(
        pl.kernel,
        mesh=_mesh(),
        compiler_params=pltpu.CompilerParams(use_tc_tiling_on_sc=False),
        out_type=jax.ShapeDtypeStruct((NB, E), jnp.bfloat16),
        scratch_types=[
            pltpu.VMEM((UC,), jnp.int32),
            pltpu.VMEM((UC,), jnp.int32),
            pltpu.VMEM((UC, E), jnp.bfloat16),
            pltpu.VMEM((UC, E), jnp.bfloat16),
            pltpu.SemaphoreType.DMA,
            pltpu.SemaphoreType.DMA,
        ],
    )
    def k(uidx_hbm, utab_hbm, a_out, idx0, idx1, rows0, rows1, sem0, sem1):
        wid = lax.axis_index("s") * NC + lax.axis_index("c")
        base_w = wid * BAGS_W

        idx_bufs = (idx0, idx1)
        row_bufs = (rows0, rows1)
        sems = (sem0, sem1)

        def issue(chunk, p):
            base = base_w + chunk * UC
            pltpu.sync_copy(uidx_hbm.at[pl.ds(base, UC)], idx_bufs[p])
            pltpu.make_async_copy(utab_hbm.at[idx_bufs[p]], row_bufs[p],
                                  sems[p]).start()

        issue(0, 0)
        issue(1, 1)
        for uc in range(UCHUNKS):
            p = uc % 2
            pltpu.make_async_copy(utab_hbm.at[idx_bufs[p]], row_bufs[p],
                                  sems[p]).wait()
            pltpu.sync_copy(row_bufs[p],
                            a_out.at[pl.ds(base_w + uc * UC, UC)])
            if uc + 2 < UCHUNKS:
                issue(uc + 2, p)

    return k(uidx, user_table)


BM = 512


def _tc_combine(a, b1, b2, Wb):
    nh = B // (2 * BM)

    def body(a_ref, b1_ref, b2_ref, w_ref, o_ref):
        i = pl.program_id(0)
        bb = jnp.where(i < nh, b1_ref[...], b2_ref[...])
        y = lax.dot_general(bb, w_ref[...], (((1,), (1,)), ((), ())),
                            preferred_element_type=jnp.float32)
        z = G * a_ref[...].astype(jnp.float32) + ((1.0 - G) / HIST) * y
        ss = jnp.sum(z * z, axis=1, keepdims=True)
        o_ref[...] = z / jnp.maximum(jnp.sqrt(ss), 1e-12)

    return pl.pallas_call(
        body,
        grid=(B // BM,),
        in_specs=[
            pl.BlockSpec((BM, EMB), lambda i: (i, 0)),
            pl.BlockSpec((BM, EMB), lambda i: (jnp.minimum(i, nh - 1), 0)),
            pl.BlockSpec((BM, EMB),
                         lambda i: (jnp.maximum(i - nh, 0), 0)),
            pl.BlockSpec((EMB, EMB), lambda i: (0, 0)),
        ],
        out_specs=pl.BlockSpec((BM, EMB), lambda i: (i, 0)),
        out_shape=jax.ShapeDtypeStruct((B, EMB), jnp.float32),
    )(a, b1, b2, Wb)


def kernel(user_idx, item_idx, user_table, item_table, W):
    itab = item_table.astype(jnp.bfloat16)
    iidx1 = item_idx[:B // 2].reshape(-1).astype(jnp.int32)
    b1_flat = _sc_item(iidx1, itab)
    iidx2 = item_idx[B // 2:].reshape(-1).astype(jnp.int32)
    b2_flat = _sc_item(iidx2, itab)
    uidx = user_idx.reshape(-1).astype(jnp.int32)
    utab = user_table.astype(jnp.bfloat16)
    a_flat = _sc_user(uidx, utab)
    a = a_flat.reshape(B, EMB)
    b1 = b1_flat.reshape(B // 2, EMB)
    b2 = b2_flat.reshape(B // 2, EMB)
    return _tc_combine(a, b1, b2, W.astype(jnp.bfloat16))


# speedup vs baseline: 1.0320x; 1.0320x over previous
import functools

import jax
import jax.numpy as jnp
from jax import lax
from jax.experimental import pallas as pl
from jax.experimental.pallas import tpu as pltpu
from jax.experimental.pallas import tpu_sc as plsc

B = 4096
F_FIELDS = 26
E = 64
HIST = 20
G = 0.8
EMB = F_FIELDS * E
NB = B * F_FIELDS

NC = 2
NS = 16
NW = NC * NS
BAGS_W = NB // NW

C = 64
CH = C * HIST
CHUNKS = BAGS_W // C
UC = 832
UCHUNKS = BAGS_W // UC

LB = 32


def _tree_sum(vals):
    while len(vals) > 1:
        nxt = [a + b for a, b in zip(vals[0::2], vals[1::2])]
        if len(vals) % 2:
            nxt.append(vals[-1])
        vals = nxt
    return vals[0]


def _mesh():
    return plsc.VectorSubcoreMesh(core_axis_name="c", subcore_axis_name="s")


def _sc_item(iidx, item_table):
    @functools.partial(
        pl.kernel,
        mesh=_mesh(),
        compiler_params=pltpu.CompilerParams(use_tc_tiling_on_sc=False),
        out_type=jax.ShapeDtypeStruct((NB, E), jnp.bfloat16),
        scratch_types=[
            pltpu.VMEM((CH,), jnp.int32),
            pltpu.VMEM((CH,), jnp.int32),
            pltpu.VMEM((CH,), jnp.int32),
            pltpu.VMEM((CH,), jnp.int32),
            pltpu.VMEM((CH, E), jnp.bfloat16),
            pltpu.VMEM((CH, E), jnp.bfloat16),
            pltpu.VMEM((C, E), jnp.bfloat16),
            pltpu.SemaphoreType.DMA,
            pltpu.SemaphoreType.DMA,
            pltpu.SemaphoreType.DMA,
            pltpu.SemaphoreType.DMA,
            pltpu.SemaphoreType.DMA,
            pltpu.SemaphoreType.DMA,
        ],
    )
    def k(iidx_hbm, itab_hbm, b_out, i0, i1, i2, i3, rows0, rows1, acc_v,
          is0, is1, is2, is3, gs0, gs1):
        wid = lax.axis_index("s") * NC + lax.axis_index("c")
        bag_base = wid * BAGS_W

        idx_bufs = (i0, i1, i2, i3)
        idx_sems = (is0, is1, is2, is3)
        row_bufs = (rows0, rows1)
        row_sems = (gs0, gs1)

        def idx_copy(chunk, q):
            return pltpu.make_async_copy(
                iidx_hbm.at[pl.ds((bag_base + chunk * C) * HIST, CH)],
                idx_bufs[q], idx_sems[q])

        def gather(q, p):
            return pltpu.make_async_copy(itab_hbm.at[idx_bufs[q]],
                                         row_bufs[p], row_sems[p])

        def reduce_chunk(chunk, p):
            rows_v = row_bufs[p]

            def bag_body(c, carry):
                r0 = c * HIST
                for j in range(E // LB):
                    s = pl.ds(j * LB, LB)
                    acc_v[c, s] = _tree_sum(
                        [rows_v[r0 + h, s] for h in range(HIST)])
                return carry

            lax.fori_loop(0, C, bag_body, 0)
            pltpu.sync_copy(acc_v, b_out.at[pl.ds(bag_base + chunk * C, C)])

        for q in range(4):
            idx_copy(q, q).start()
        idx_copy(0, 0).wait()
        gather(0, 0).start()

        def quad_body(t, carry):
            for i in range(4):
                c = 4 * t + i
                q = i
                p = i % 2
                gather(q, p).wait()

                @pl.when(c + 4 < CHUNKS)
                def _():
                    idx_copy(c + 4, q).start()

                @pl.when(c + 1 < CHUNKS)
                def _():
                    idx_copy(c + 1, (q + 1) % 4).wait()
                    gather((q + 1) % 4, (p + 1) % 2).start()

                reduce_chunk(c, p)
            return carry

        lax.fori_loop(0, CHUNKS // 4, quad_body, 0)

    return k(iidx, item_table)


def _sc_user(uidx, user_table):
    @functools.partial(
        pl.kernel,
        mesh=_mesh(),
        compiler_params=pltpu.CompilerParams(use_tc_tiling_on_sc=False),
        out_type=jax.ShapeDtypeStruct((NB, E), jnp.bfloat16),
        scratch_types=[
            pltpu.VMEM((UC,), jnp.int32),
            pltpu.VMEM((UC,), jnp.int32),
            pltpu.VMEM((UC, E), jnp.bfloat16),
            pltpu.VMEM((UC, E), jnp.bfloat16),
            pltpu.SemaphoreType.DMA,
            pltpu.SemaphoreType.DMA,
        ],
    )
    def k(uidx_hbm, utab_hbm, a_out, idx0, idx1, rows0, rows1, sem0, sem1):
        wid = lax.axis_index("s") * NC + lax.axis_index("c")
        base_w = wid * BAGS_W

        idx_bufs = (idx0, idx1)
        row_bufs = (rows0, rows1)
        sems = (sem0, sem1)

        def issue(chunk, p):
            base = base_w + chunk * UC
            pltpu.sync_copy(uidx_hbm.at[pl.ds(base, UC)], idx_bufs[p])
            pltpu.make_async_copy(utab_hbm.at[idx_bufs[p]], row_bufs[p],
                                  sems[p]).start()

        issue(0, 0)
        issue(1, 1)
        for uc in range(UCHUNKS):
            p = uc % 2
            pltpu.make_async_copy(utab_hbm.at[idx_bufs[p]], row_bufs[p],
                                  sems[p]).wait()
            pltpu.sync_copy(row_bufs[p],
                            a_out.at[pl.ds(base_w + uc * UC, UC)])
            if uc + 2 < UCHUNKS:
                issue(uc + 2, p)

    return k(uidx, user_table)


BM = 512


def _tc_combine(a, bsum, Wb):
    def body(a_ref, b_ref, w_ref, o_ref):
        y = lax.dot_general(b_ref[...], w_ref[...], (((1,), (1,)), ((), ())),
                            preferred_element_type=jnp.float32)
        z = G * a_ref[...].astype(jnp.float32) + ((1.0 - G) / HIST) * y
        ss = jnp.sum(z * z, axis=1, keepdims=True)
        o_ref[...] = z / jnp.maximum(jnp.sqrt(ss), 1e-12)

    return pl.pallas_call(
        body,
        grid=(B // BM,),
        in_specs=[
            pl.BlockSpec((BM, EMB), lambda i: (i, 0)),
            pl.BlockSpec((BM, EMB), lambda i: (i, 0)),
            pl.BlockSpec((EMB, EMB), lambda i: (0, 0)),
        ],
        out_specs=pl.BlockSpec((BM, EMB), lambda i: (i, 0)),
        out_shape=jax.ShapeDtypeStruct((B, EMB), jnp.float32),
    )(a, bsum, Wb)


def kernel(user_idx, item_idx, user_table, item_table, W):
    iidx = item_idx.reshape(-1).astype(jnp.int32)
    itab = item_table.astype(jnp.bfloat16)
    bsum_flat = _sc_item(iidx, itab)
    uidx = user_idx.reshape(-1).astype(jnp.int32)
    utab = user_table.astype(jnp.bfloat16)
    a_flat = _sc_user(uidx, utab)
    a = a_flat.reshape(B, EMB)
    bsum = bsum_flat.reshape(B, EMB)
    return _tc_combine(a, bsum, W.astype(jnp.bfloat16))
